# baseline (speedup 1.0000x reference)
import jax
import jax.numpy as jnp
from jax.experimental import pallas as pl
from jax.experimental.pallas import tpu as pltpu

S = 2048
D = 1024
A = 72
P = 12
T = 600
EPS = 1e-5
ACC_COLS = 2688


def _sp(x):
    return jnp.maximum(x, 0.0) + jnp.log1p(jnp.exp(-jnp.abs(x)))


def _ln_matmul_kernel(hs_ref, g_ref, beta_ref, W_ref, bias_ref, out_ref):
    x = hs_ref[...]
    mean = jnp.mean(x, axis=1, keepdims=True)
    xc = x - mean
    var = jnp.mean(xc * xc, axis=1, keepdims=True)
    normed = xc * jax.lax.rsqrt(var + EPS) * g_ref[...] + beta_ref[...]
    out_ref[...] = jax.lax.dot_general(
        W_ref[...], normed, (((1,), (1,)), ((), ())),
        preferred_element_type=jnp.float32,
        precision=jax.lax.Precision.DEFAULT) + bias_ref[...]


def _band_kernel(np_ref,
                 kin_ref, ab_ref, as_ref, gb_ref, gs_ref,
                 acc_ref, u_ref, v_ref, ut_ref, vt_ref,
                 a_ref, b_ref, at_ref, bt_ref):
    np0 = np_ref[0 * A:1 * A, :]
    np1 = np_ref[1 * A:2 * A, :]
    np2 = np_ref[2 * A:3 * A, :]
    np3 = np_ref[3 * A:4 * A, :]
    c = np_ref[4 * A:5 * A, :]
    cth = np_ref[5 * A:6 * A, :]
    phi = np_ref[6 * A:7 * A, :]
    phith = np_ref[7 * A:8 * A, :]
    ab_ref[...] = np_ref[8 * A:9 * A, :]
    as_ref[...] = _sp(np_ref[9 * A:10 * A, :])
    gb_ref[...] = np_ref[10 * A:11 * A, :]
    gs_ref[...] = _sp(np_ref[11 * A:12 * A, :])

    d = _sp(np1)
    k = d * d / 4.0 + _sp(np0)
    om = jnp.sqrt(k * 4.0 - d * d) / 2.0
    dec = jnp.exp(-d / 2.0)
    a_ref[...] = dec * jnp.cos(om)
    b_ref[...] = dec * jnp.sin(om)
    u_ref[...] = c * jnp.sin(phi)
    v_ref[...] = c * jnp.cos(phi)

    dth = _sp(np3)
    kth = dth * dth / 4.0 + _sp(np2)
    omt = jnp.sqrt(kth * 4.0 - dth * dth) / 2.0
    dect = jnp.exp(-dth / 2.0)
    at_ref[...] = dect * jnp.cos(omt)
    bt_ref[...] = dect * jnp.sin(omt)
    ut_ref[...] = cth * jnp.sin(phith)
    vt_ref[...] = cth * jnp.cos(phith)

    acc_ref[...] = jnp.zeros((A, ACC_COLS), jnp.float32)
    G = 8
    AB = 8
    SC = 512
    zpad = jnp.zeros((AB, 128), jnp.float32)
    NCH = (A // AB) * (S // SC)

    def body(i, carry):
        g = i // NCH
        c = i - g * NCH
        ai = c // (S // SC)
        si = c - ai * (S // SC)
        row = pl.multiple_of(ai * AB, AB)
        col = pl.multiple_of(si * SC, SC)
        t0 = g * G
        q = t0 // 128
        r = t0 - q * 128
        base = pl.multiple_of(q * 128 + si * SC, 128)
        rows = pl.ds(row, AB)
        cols = pl.ds(col, SC)
        u = u_ref[rows, cols]
        v = v_ref[rows, cols]
        ut = ut_ref[rows, cols]
        vt = vt_ref[rows, cols]
        a = a_ref[rows, cols]
        b = b_ref[rows, cols]
        at = at_ref[rows, cols]
        bt = bt_ref[rows, cols]
        wide = jnp.concatenate([u + ut, zpad], axis=1)
        for j in range(1, G):
            un = a * u + b * v
            v = a * v - b * u
            u = un
            utn = at * ut + bt * vt
            vt = at * vt - bt * ut
            ut = utn
            wide = wide + pltpu.roll(
                jnp.concatenate([u + ut, zpad], axis=1), j, 1)
        un = a * u + b * v
        v_ref[rows, cols] = a * v - b * u
        u_ref[rows, cols] = un
        utn = at * ut + bt * vt
        vt_ref[rows, cols] = at * vt - bt * ut
        ut_ref[rows, cols] = utn
        acc_ref[rows, pl.ds(base, SC + 128)] += pltpu.roll(wide, r, 1)
        return carry

    jax.lax.fori_loop(0, (T // G) * NCH, body, 0)
    kin_ref[...] = acc_ref[:, 0:S]


def kernel(hidden_states, ln_gamma, ln_beta, W, b):
    hs = hidden_states[0]
    g = ln_gamma.reshape(1, D)
    beta = ln_beta.reshape(1, D)
    bias = b.reshape(A * P, 1)

    SB = 256
    npar_t = pl.pallas_call(
        _ln_matmul_kernel,
        grid=(S // SB,),
        in_specs=[
            pl.BlockSpec((SB, D), lambda i: (i, 0)),
            pl.BlockSpec((1, D), lambda i: (0, 0)),
            pl.BlockSpec((1, D), lambda i: (0, 0)),
            pl.BlockSpec((A * P, D), lambda i: (0, 0)),
            pl.BlockSpec((A * P, 1), lambda i: (0, 0)),
        ],
        out_specs=pl.BlockSpec((A * P, SB), lambda i: (0, i)),
        out_shape=jax.ShapeDtypeStruct((A * P, S), jnp.float32),
    )(hs, g, beta, W, bias)

    out_sd = jax.ShapeDtypeStruct((A, S), jnp.float32)
    kin, ab, as_, gb, gs = pl.pallas_call(
        _band_kernel,
        out_shape=[out_sd] * 5,
        scratch_shapes=[pltpu.VMEM((A, ACC_COLS), jnp.float32)]
        + [pltpu.VMEM((A, S), jnp.float32)] * 8,
    )(npar_t)
    return kin, ab, as_, gb, gs


# speedup vs baseline: 2.4219x; 2.4219x over previous
import jax
import jax.numpy as jnp
from jax.experimental import pallas as pl
from jax.experimental.pallas import tpu as pltpu

S = 2048
D = 1024
A = 72
P = 12
T = 600
EPS = 1e-5
ACC_COLS = 2688


def _sp(x):
    return jnp.maximum(x, 0.0) + jnp.log1p(jnp.exp(-jnp.abs(x)))


def _ln_matmul_kernel(hs_ref, g_ref, beta_ref, W_ref, bias_ref, out_ref):
    x = hs_ref[...]
    mean = jnp.mean(x, axis=1, keepdims=True)
    xc = x - mean
    var = jnp.mean(xc * xc, axis=1, keepdims=True)
    normed = xc * jax.lax.rsqrt(var + EPS) * g_ref[...] + beta_ref[...]
    out_ref[...] = jax.lax.dot_general(
        W_ref[...], normed, (((1,), (1,)), ((), ())),
        preferred_element_type=jnp.float32,
        precision=jax.lax.Precision.DEFAULT) + bias_ref[...]


def _band_kernel(np_ref,
                 kin_ref, ab_ref, as_ref, gb_ref, gs_ref,
                 acc_ref, u_ref, v_ref, ut_ref, vt_ref,
                 a_ref, b_ref, at_ref, bt_ref):
    np0 = np_ref[0 * A:1 * A, :]
    np1 = np_ref[1 * A:2 * A, :]
    np2 = np_ref[2 * A:3 * A, :]
    np3 = np_ref[3 * A:4 * A, :]
    c = np_ref[4 * A:5 * A, :]
    cth = np_ref[5 * A:6 * A, :]
    phi = np_ref[6 * A:7 * A, :]
    phith = np_ref[7 * A:8 * A, :]
    ab_ref[...] = np_ref[8 * A:9 * A, :]
    as_ref[...] = _sp(np_ref[9 * A:10 * A, :])
    gb_ref[...] = np_ref[10 * A:11 * A, :]
    gs_ref[...] = _sp(np_ref[11 * A:12 * A, :])

    d = _sp(np1)
    k = d * d / 4.0 + _sp(np0)
    om = jnp.sqrt(k * 4.0 - d * d) / 2.0
    dec = jnp.exp(-d / 2.0)
    cosom = jnp.cos(om)
    sinom = jnp.sin(om)
    a_ref[...] = 2.0 * dec * cosom
    b_ref[...] = dec * dec
    u0 = c * jnp.sin(phi)
    cv0 = c * jnp.cos(phi)
    u_ref[...] = u0
    v_ref[...] = jnp.exp(d / 2.0) * (u0 * cosom - cv0 * sinom)

    dth = _sp(np3)
    kth = dth * dth / 4.0 + _sp(np2)
    omt = jnp.sqrt(kth * 4.0 - dth * dth) / 2.0
    dect = jnp.exp(-dth / 2.0)
    cosomt = jnp.cos(omt)
    sinomt = jnp.sin(omt)
    at_ref[...] = 2.0 * dect * cosomt
    bt_ref[...] = dect * dect
    ut0 = cth * jnp.sin(phith)
    cvt0 = cth * jnp.cos(phith)
    ut_ref[...] = ut0
    vt_ref[...] = jnp.exp(dth / 2.0) * (ut0 * cosomt - cvt0 * sinomt)

    acc_ref[...] = jnp.zeros((A, ACC_COLS), jnp.float32)
    zpad = jnp.zeros((A, 128), jnp.float32)
    G = 8

    def body(g, carry):
        t0 = g * G
        q = t0 // 128
        r = t0 - q * 128
        base = pl.multiple_of(q * 128, 128)
        u = u_ref[...]
        v = v_ref[...]
        ut = ut_ref[...]
        vt = vt_ref[...]
        a = a_ref[...]
        b = b_ref[...]
        at = at_ref[...]
        bt = bt_ref[...]
        wide = jnp.concatenate([u + ut, zpad], axis=1)
        for j in range(1, G):
            un = a * u - b * v
            v = u
            u = un
            utn = at * ut - bt * vt
            vt = ut
            ut = utn
            wide = wide + pltpu.roll(
                jnp.concatenate([u + ut, zpad], axis=1), j, 1)
        u_ref[...] = a * u - b * v
        v_ref[...] = u
        ut_ref[...] = at * ut - bt * vt
        vt_ref[...] = ut
        acc_ref[:, pl.ds(base, S + 128)] += pltpu.roll(wide, r, 1)
        return carry

    jax.lax.fori_loop(0, T // G, body, 0)
    kin_ref[...] = acc_ref[:, 0:S]


def kernel(hidden_states, ln_gamma, ln_beta, W, b):
    hs = hidden_states[0]
    g = ln_gamma.reshape(1, D)
    beta = ln_beta.reshape(1, D)
    bias = b.reshape(A * P, 1)

    SB = 256
    npar_t = pl.pallas_call(
        _ln_matmul_kernel,
        grid=(S // SB,),
        in_specs=[
            pl.BlockSpec((SB, D), lambda i: (i, 0)),
            pl.BlockSpec((1, D), lambda i: (0, 0)),
            pl.BlockSpec((1, D), lambda i: (0, 0)),
            pl.BlockSpec((A * P, D), lambda i: (0, 0)),
            pl.BlockSpec((A * P, 1), lambda i: (0, 0)),
        ],
        out_specs=pl.BlockSpec((A * P, SB), lambda i: (0, i)),
        out_shape=jax.ShapeDtypeStruct((A * P, S), jnp.float32),
    )(hs, g, beta, W, bias)

    out_sd = jax.ShapeDtypeStruct((A, S), jnp.float32)
    kin, ab, as_, gb, gs = pl.pallas_call(
        _band_kernel,
        out_shape=[out_sd] * 5,
        scratch_shapes=[pltpu.VMEM((A, ACC_COLS), jnp.float32)]
        + [pltpu.VMEM((A, S), jnp.float32)] * 8,
    )(npar_t)
    return kin, ab, as_, gb, gs


# speedup vs baseline: 2.4712x; 1.0204x over previous
import jax
import jax.numpy as jnp
from jax.experimental import pallas as pl
from jax.experimental.pallas import tpu as pltpu

S = 2048
D = 1024
A = 72
P = 12
T = 600
EPS = 1e-5
ACC_COLS = 2688


def _sp(x):
    return jnp.maximum(x, 0.0) + jnp.log1p(jnp.exp(-jnp.abs(x)))


def _ln_matmul_kernel(hs_ref, g_ref, beta_ref, W_ref, bias_ref, out_ref):
    x = hs_ref[...]
    mean = jnp.mean(x, axis=1, keepdims=True)
    xc = x - mean
    var = jnp.mean(xc * xc, axis=1, keepdims=True)
    normed = xc * jax.lax.rsqrt(var + EPS) * g_ref[...] + beta_ref[...]
    out_ref[...] = jax.lax.dot_general(
        W_ref[...], normed, (((1,), (1,)), ((), ())),
        preferred_element_type=jnp.float32,
        precision=jax.lax.Precision.DEFAULT) + bias_ref[...]


def _band_kernel(np_ref,
                 kin_ref, ab_ref, as_ref, gb_ref, gs_ref,
                 acc_ref, u_ref, v_ref, ut_ref, vt_ref,
                 a_ref, b_ref, at_ref, bt_ref):
    np0 = np_ref[0 * A:1 * A, :]
    np1 = np_ref[1 * A:2 * A, :]
    np2 = np_ref[2 * A:3 * A, :]
    np3 = np_ref[3 * A:4 * A, :]
    c = np_ref[4 * A:5 * A, :]
    cth = np_ref[5 * A:6 * A, :]
    phi = np_ref[6 * A:7 * A, :]
    phith = np_ref[7 * A:8 * A, :]
    ab_ref[...] = np_ref[8 * A:9 * A, :]
    as_ref[...] = _sp(np_ref[9 * A:10 * A, :])
    gb_ref[...] = np_ref[10 * A:11 * A, :]
    gs_ref[...] = _sp(np_ref[11 * A:12 * A, :])

    d = _sp(np1)
    k = d * d / 4.0 + _sp(np0)
    om = jnp.sqrt(k * 4.0 - d * d) / 2.0
    dec = jnp.exp(-d / 2.0)
    cosom = jnp.cos(om)
    sinom = jnp.sin(om)
    a_ref[...] = 2.0 * dec * cosom
    b_ref[...] = dec * dec
    u0 = c * jnp.sin(phi)
    cv0 = c * jnp.cos(phi)
    u_ref[...] = u0
    v_ref[...] = jnp.exp(d / 2.0) * (u0 * cosom - cv0 * sinom)

    dth = _sp(np3)
    kth = dth * dth / 4.0 + _sp(np2)
    omt = jnp.sqrt(kth * 4.0 - dth * dth) / 2.0
    dect = jnp.exp(-dth / 2.0)
    cosomt = jnp.cos(omt)
    sinomt = jnp.sin(omt)
    at_ref[...] = 2.0 * dect * cosomt
    bt_ref[...] = dect * dect
    ut0 = cth * jnp.sin(phith)
    cvt0 = cth * jnp.cos(phith)
    ut_ref[...] = ut0
    vt_ref[...] = jnp.exp(dth / 2.0) * (ut0 * cosomt - cvt0 * sinomt)

    acc_ref[...] = jnp.zeros((A, ACC_COLS), jnp.float32)
    zpad = jnp.zeros((A, 128), jnp.float32)

    def make_body(G):
        return lambda g, carry: _group(g, carry, G)

    def _group(g, carry, G):
        t0 = g * G
        q = t0 // 128
        r = t0 - q * 128
        base = pl.multiple_of(q * 128, 128)
        u = u_ref[...]
        v = v_ref[...]
        ut = ut_ref[...]
        vt = vt_ref[...]
        a = a_ref[...]
        b = b_ref[...]
        at = at_ref[...]
        bt = bt_ref[...]
        wide = jnp.concatenate([u + ut, zpad], axis=1)
        for j in range(1, G):
            un = a * u - b * v
            v = u
            u = un
            utn = at * ut - bt * vt
            vt = ut
            ut = utn
            wide = wide + pltpu.roll(
                jnp.concatenate([u + ut, zpad], axis=1), j, 1)
        u_ref[...] = a * u - b * v
        v_ref[...] = u
        ut_ref[...] = at * ut - bt * vt
        vt_ref[...] = ut
        acc_ref[:, pl.ds(base, S + 128)] += pltpu.roll(wide, r, 1)
        return carry

    jax.lax.fori_loop(0, 37, make_body(16), 0)
    jax.lax.fori_loop(74, 75, make_body(8), 0)
    kin_ref[...] = acc_ref[:, 0:S]


def kernel(hidden_states, ln_gamma, ln_beta, W, b):
    hs = hidden_states[0]
    g = ln_gamma.reshape(1, D)
    beta = ln_beta.reshape(1, D)
    bias = b.reshape(A * P, 1)

    SB = 256
    npar_t = pl.pallas_call(
        _ln_matmul_kernel,
        grid=(S // SB,),
        in_specs=[
            pl.BlockSpec((SB, D), lambda i: (i, 0)),
            pl.BlockSpec((1, D), lambda i: (0, 0)),
            pl.BlockSpec((1, D), lambda i: (0, 0)),
            pl.BlockSpec((A * P, D), lambda i: (0, 0)),
            pl.BlockSpec((A * P, 1), lambda i: (0, 0)),
        ],
        out_specs=pl.BlockSpec((A * P, SB), lambda i: (0, i)),
        out_shape=jax.ShapeDtypeStruct((A * P, S), jnp.float32),
    )(hs, g, beta, W, bias)

    out_sd = jax.ShapeDtypeStruct((A, S), jnp.float32)
    kin, ab, as_, gb, gs = pl.pallas_call(
        _band_kernel,
        out_shape=[out_sd] * 5,
        scratch_shapes=[pltpu.VMEM((A, ACC_COLS), jnp.float32)]
        + [pltpu.VMEM((A, S), jnp.float32)] * 8,
    )(npar_t)
    return kin, ab, as_, gb, gs
